# baseline (speedup 1.0000x reference)
import functools

import jax
import jax.numpy as jnp
from jax import lax
from jax.experimental import pallas as pl
from jax.experimental.pallas import tpu as pltpu
from jax.experimental.pallas import tpu_sc as plsc

N_EXP = 16
N_TOK = 2048
D_MODEL = 1024
D_FF = 512
TILE = 256
NT = N_TOK // TILE
NW = 32
EPT = (N_TOK * 2) // NW
TPW = N_TOK // NW


def _router_body(x_ref, wgt_ref, e1_ref, e2_ref, r1_ref, r2_ref,
                 p1_ref, p2_ref, cnt_ref):
    x = x_ref[...]
    logits = jnp.dot(x, wgt_ref[...], preferred_element_type=jnp.float32)
    m = jnp.max(logits, axis=-1, keepdims=True)
    p = jnp.exp(logits - m)
    p = p / jnp.sum(p, axis=-1, keepdims=True)
    idx = lax.broadcasted_iota(jnp.int32, p.shape, 1)
    big = jnp.int32(N_EXP + 1)
    m1 = jnp.max(p, axis=-1, keepdims=True)
    i1 = jnp.min(jnp.where(p >= m1, idx, big), axis=-1, keepdims=True)
    pm = jnp.where(idx == i1, -jnp.inf, p)
    m2 = jnp.max(pm, axis=-1, keepdims=True)
    i2 = jnp.min(jnp.where(pm >= m2, idx, big), axis=-1, keepdims=True)

    oh1 = (idx == i1).astype(jnp.float32)
    oh2 = (idx == i2).astype(jnp.float32)
    oh = oh1 + oh2
    s = oh
    sh = 1
    while sh < N_TOK:
        s = s + jnp.concatenate(
            [jnp.zeros((sh, N_EXP), jnp.float32), s[:-sh, :]], axis=0)
        sh *= 2
    excl = s - oh
    r1 = jnp.sum(excl * oh1, axis=-1, keepdims=True)
    r2 = jnp.sum((excl + oh1) * oh2, axis=-1, keepdims=True)

    e1_ref[...] = i1
    e2_ref[...] = i2
    r1_ref[...] = r1.astype(jnp.int32)
    r2_ref[...] = r2.astype(jnp.int32)
    p1_ref[...] = m1
    p2_ref[...] = m2
    cnt_ref[...] = jnp.sum(oh, axis=0, keepdims=True).astype(jnp.int32)


def _router(x_flat, WgT):
    i32 = jnp.int32
    f32 = jnp.float32
    outs = pl.pallas_call(
        _router_body,
        out_shape=[
            jax.ShapeDtypeStruct((N_TOK, 1), i32),
            jax.ShapeDtypeStruct((N_TOK, 1), i32),
            jax.ShapeDtypeStruct((N_TOK, 1), i32),
            jax.ShapeDtypeStruct((N_TOK, 1), i32),
            jax.ShapeDtypeStruct((N_TOK, 1), f32),
            jax.ShapeDtypeStruct((N_TOK, 1), f32),
            jax.ShapeDtypeStruct((1, N_EXP), i32),
        ],
    )(x_flat, WgT)
    return outs


def _dispatch_sc(x_flat, ek, rk):
    mesh = plsc.VectorSubcoreMesh(core_axis_name="c", subcore_axis_name="s")

    @functools.partial(
        pl.kernel,
        mesh=mesh,
        out_type=jax.ShapeDtypeStruct((N_EXP * N_TOK, D_MODEL), jnp.float32),
        scratch_types=[
            pltpu.VMEM((EPT,), jnp.int32),
            pltpu.VMEM((EPT,), jnp.int32),
            pltpu.VMEM((EPT // 2,), jnp.int32),
            pltpu.VMEM((EPT // 2,), jnp.int32),
            pltpu.VMEM((EPT // 2, D_MODEL), jnp.float32),
        ],
    )
    def k(x_hbm, ek_hbm, rk_hbm, xg_hbm, ev, rv, dva, dvb, rows):
        wid = lax.axis_index("s") * 2 + lax.axis_index("c")
        kk = wid & 1
        mm = wid >> 1
        base = mm * EPT
        pltpu.sync_copy(ek_hbm.at[kk, pl.ds(base, EPT)], ev)
        pltpu.sync_copy(rk_hbm.at[kk, pl.ds(base, EPT)], rv)
        for c in range(EPT // 16):
            e16 = ev[pl.ds(c * 16, 16)]
            r16 = rv[pl.ds(c * 16, 16)]
            d16 = e16 * N_TOK + r16
            half = c // (EPT // 32)
            off = (c % (EPT // 32)) * 16
            if half == 0:
                dva[pl.ds(off, 16)] = d16
            else:
                dvb[pl.ds(off, 16)] = d16
        pltpu.sync_copy(x_hbm.at[pl.ds(base, EPT // 2)], rows)
        pltpu.sync_copy(rows, xg_hbm.at[dva])
        pltpu.sync_copy(x_hbm.at[pl.ds(base + EPT // 2, EPT // 2)], rows)
        pltpu.sync_copy(rows, xg_hbm.at[dvb])

    return k(x_flat, ek, rk)


def _gmm_body(cnt_ref, xg_ref, w1_ref, w3_ref, w2_ref, yg_ref):
    e = pl.program_id(0)
    t = pl.program_id(1)

    @pl.when(t * TILE < cnt_ref[e])
    def _compute():
        xg = xg_ref[...]
        h1 = jnp.dot(xg, w1_ref[0], preferred_element_type=jnp.float32)
        h3 = jnp.dot(xg, w3_ref[0], preferred_element_type=jnp.float32)
        h = (h1 / (1.0 + jnp.exp(-h1))) * h3
        yg_ref[...] = jnp.dot(h, w2_ref[0], preferred_element_type=jnp.float32)


def _row_blk(e, t, cnt):
    mt = (cnt[e] + TILE - 1) // TILE
    tc = jnp.minimum(t, jnp.maximum(mt - 1, 0))
    return e * NT + tc, 0


def _gmm(counts, xg, W1, W3, W2):
    grid_spec = pltpu.PrefetchScalarGridSpec(
        num_scalar_prefetch=1,
        grid=(N_EXP, NT),
        in_specs=[
            pl.BlockSpec((TILE, D_MODEL), _row_blk),
            pl.BlockSpec((1, D_MODEL, D_FF), lambda e, t, cnt: (e, 0, 0)),
            pl.BlockSpec((1, D_MODEL, D_FF), lambda e, t, cnt: (e, 0, 0)),
            pl.BlockSpec((1, D_FF, D_MODEL), lambda e, t, cnt: (e, 0, 0)),
        ],
        out_specs=pl.BlockSpec((TILE, D_MODEL), _row_blk),
    )
    return pl.pallas_call(
        _gmm_body,
        grid_spec=grid_spec,
        out_shape=jax.ShapeDtypeStruct((N_EXP * N_TOK, D_MODEL), jnp.float32),
        compiler_params=pltpu.CompilerParams(
            dimension_semantics=("arbitrary", "arbitrary"),
        ),
    )(counts, xg, W1, W3, W2)


def _gatherback_sc(yg, ek, rk):
    mesh = plsc.VectorSubcoreMesh(core_axis_name="c", subcore_axis_name="s")

    @functools.partial(
        pl.kernel,
        mesh=mesh,
        out_type=[
            jax.ShapeDtypeStruct((N_TOK, D_MODEL), jnp.float32),
            jax.ShapeDtypeStruct((N_TOK, D_MODEL), jnp.float32),
        ],
        scratch_types=[
            pltpu.VMEM((TPW,), jnp.int32),
            pltpu.VMEM((TPW,), jnp.int32),
            pltpu.VMEM((TPW,), jnp.int32),
            pltpu.VMEM((TPW,), jnp.int32),
            pltpu.VMEM((TPW // 2, D_MODEL), jnp.float32),
        ],
    )
    def k(yg_hbm, ek_hbm, rk_hbm, y1_hbm, y2_hbm,
          i1v, i2v, evs, rvs, rows):
        wid = lax.axis_index("s") * 2 + lax.axis_index("c")
        base = wid * TPW
        pltpu.sync_copy(ek_hbm.at[0, pl.ds(base, TPW)], evs)
        pltpu.sync_copy(rk_hbm.at[0, pl.ds(base, TPW)], rvs)
        for c in range(TPW // 16):
            i1v[pl.ds(c * 16, 16)] = (
                evs[pl.ds(c * 16, 16)] * N_TOK + rvs[pl.ds(c * 16, 16)])
        pltpu.sync_copy(ek_hbm.at[1, pl.ds(base, TPW)], evs)
        pltpu.sync_copy(rk_hbm.at[1, pl.ds(base, TPW)], rvs)
        for c in range(TPW // 16):
            i2v[pl.ds(c * 16, 16)] = (
                evs[pl.ds(c * 16, 16)] * N_TOK + rvs[pl.ds(c * 16, 16)])

        half = TPW // 2
        for c in range(2):
            pltpu.sync_copy(yg_hbm.at[i1v.at[pl.ds(c * half, half)]], rows)
            pltpu.sync_copy(rows, y1_hbm.at[pl.ds(base + c * half, half)])
            pltpu.sync_copy(yg_hbm.at[i2v.at[pl.ds(c * half, half)]], rows)
            pltpu.sync_copy(rows, y2_hbm.at[pl.ds(base + c * half, half)])

    return k(yg, ek, rk)


def _scale_body(y1_ref, y2_ref, p1_ref, p2_ref, y_ref):
    y_ref[...] = p1_ref[...] * y1_ref[...] + p2_ref[...] * y2_ref[...]


def _scale_add(y1, y2, p1, p2):
    blk = 256
    nb = N_TOK // blk
    return pl.pallas_call(
        _scale_body,
        grid=(nb,),
        in_specs=[
            pl.BlockSpec((blk, D_MODEL), lambda i: (i, 0)),
            pl.BlockSpec((blk, D_MODEL), lambda i: (i, 0)),
            pl.BlockSpec((blk, 1), lambda i: (i, 0)),
            pl.BlockSpec((blk, 1), lambda i: (i, 0)),
        ],
        out_specs=pl.BlockSpec((blk, D_MODEL), lambda i: (i, 0)),
        out_shape=jax.ShapeDtypeStruct((N_TOK, D_MODEL), jnp.float32),
    )(y1, y2, p1, p2)


def kernel(x, Wg, W1, W3, W2):
    Bs, Ts, C = x.shape
    x_flat = x.reshape(-1, C)

    e1, e2, r1, r2, p1, p2, cnt = _router(x_flat, Wg.T)
    ek = jnp.concatenate([e1.reshape(1, -1), e2.reshape(1, -1)], axis=0)
    rk = jnp.concatenate([r1.reshape(1, -1), r2.reshape(1, -1)], axis=0)
    counts = cnt.reshape(N_EXP)

    xg = _dispatch_sc(x_flat, ek, rk)
    yg = _gmm(counts, xg, W1, W3, W2)
    y1, y2 = _gatherback_sc(yg, ek, rk)
    y = _scale_add(y1, y2, p1, p2)
    return y.reshape(Bs, Ts, C)
